# Initial kernel scaffold; baseline (speedup 1.0000x reference)
import jax
import jax.numpy as jnp
from jax.experimental import pallas as pl


def kernel(x, edge_index, W1, b1, W2, b2):
    raise NotImplementedError("write your pallas kernel here")


# speedup vs baseline: 10.7331x; 10.7331x over previous
import functools

import jax
import jax.numpy as jnp
from jax import lax
from jax.experimental import pallas as pl
from jax.experimental.pallas import tpu as pltpu
from jax.experimental.pallas import tpu_sc as plsc

N_NODES = 10000
N_EDGES = 320000
D_IN = 128
D_HID = 128
D_OUT = 47
D_OUT_PAD = 48

NC = 2
NS = 16
NW = NC * NS

B_EDGE = 128
E_PAD = 327680
N_CHUNKS = E_PAD // B_EDGE
CPT = N_CHUNKS // NW
ACC_ROWS = 10240
DEG_ROWS = 10240

_mesh = plsc.VectorSubcoreMesh(core_axis_name="c", subcore_axis_name="s",
                               num_cores=NC, num_subcores=NS)


def _deg_body(dst_hbm, zeros_hbm, out_hbm, dst_v, ones_v, acc):
    cid = lax.axis_index("c")
    sid = lax.axis_index("s")
    wid = sid * NC + cid
    stripe = DEG_ROWS // NS
    pltpu.sync_copy(dst_hbm.at[pl.ds(wid * CPT, CPT)], dst_v)
    for i in range(B_EDGE // 16):
        ones_v[pl.ds(16 * i, 16)] = jnp.ones((16,), jnp.float32)
    pltpu.sync_copy(zeros_hbm, acc.at[pl.ds(sid * stripe, stripe)])
    plsc.subcore_barrier()

    def body(g, carry):
        pltpu.sync_copy(ones_v, acc.at[dst_v.at[g]], add=True)
        return carry

    lax.fori_loop(0, CPT, body, 0)
    plsc.subcore_barrier()
    pltpu.sync_copy(acc.at[pl.ds(sid * stripe, stripe)],
                    out_hbm.at[cid, pl.ds(sid * stripe, stripe)])


_deg_kernel = functools.partial(
    pl.kernel,
    _deg_body,
    out_type=jax.ShapeDtypeStruct((NC, DEG_ROWS), jnp.float32),
    mesh=_mesh,
    scratch_types=[
        pltpu.VMEM((CPT, B_EDGE), jnp.int32),
        pltpu.VMEM((B_EDGE,), jnp.float32),
        pltpu.VMEM_SHARED((DEG_ROWS,), jnp.float32),
    ],
)()


def _make_agg(d_feat):
    zstripe = ACC_ROWS // NS
    ostripe = ACC_ROWS // NS

    def body(y_hbm, src_hbm, dst_hbm, zeros_hbm, out_hbm,
             src_v, dst_v, buf, acc, sem):
        cid = lax.axis_index("c")
        sid = lax.axis_index("s")
        wid = sid * NC + cid
        pltpu.sync_copy(src_hbm.at[pl.ds(wid * CPT, CPT)], src_v)
        pltpu.sync_copy(dst_hbm.at[pl.ds(wid * CPT, CPT)], dst_v)
        pltpu.sync_copy(zeros_hbm, acc.at[pl.ds(sid * zstripe, zstripe)])
        plsc.subcore_barrier()

        def chunk(g, carry):
            pltpu.async_copy(y_hbm.at[src_v.at[g]], buf, sem).wait()
            pltpu.sync_copy(buf, acc.at[dst_v.at[g]], add=True)
            return carry

        lax.fori_loop(0, CPT, chunk, 0)
        plsc.subcore_barrier()
        pltpu.sync_copy(acc.at[pl.ds(sid * ostripe, ostripe)],
                        out_hbm.at[cid, pl.ds(sid * ostripe, ostripe)])

    return functools.partial(
        pl.kernel,
        body,
        out_type=jax.ShapeDtypeStruct((NC, ACC_ROWS, d_feat), jnp.float32),
        mesh=_mesh,
        scratch_types=[
            pltpu.VMEM((CPT, B_EDGE), jnp.int32),
            pltpu.VMEM((CPT, B_EDGE), jnp.int32),
            pltpu.VMEM((B_EDGE, d_feat), jnp.float32),
            pltpu.VMEM_SHARED((ACC_ROWS, d_feat), jnp.float32),
            pltpu.SemaphoreType.DMA,
        ],
        compiler_params=pltpu.CompilerParams(
            use_tc_tiling_on_sc=(d_feat % 128 == 0)),
    )()


_agg128 = _make_agg(D_HID)
_agg48 = _make_agg(D_OUT_PAD)


def _tc_pre(x_ref, w1_ref, degt_ref, y_ref):
    deg = degt_ref[:, 0:1] + degt_ref[:, 1:2] + 1.0
    dinv = lax.rsqrt(deg)
    y_ref[...] = jnp.dot(x_ref[...], w1_ref[...],
                         preferred_element_type=jnp.float32) * dinv


def _tc_mid(a1_ref, y1_ref, degt_ref, w2_ref, b1_ref, y2_ref):
    deg = degt_ref[:, 0:1] + degt_ref[:, 1:2] + 1.0
    dinv = lax.rsqrt(deg)
    h = dinv * (a1_ref[0, :N_NODES] + a1_ref[1, :N_NODES] + y1_ref[...]) + b1_ref[...]
    h = jnp.maximum(h, 0.0)
    y2_ref[...] = jnp.dot(h, w2_ref[...],
                          preferred_element_type=jnp.float32) * dinv


def _tc_post(a2_ref, y2_ref, degt_ref, b2_ref, out_ref):
    deg = degt_ref[:, 0:1] + degt_ref[:, 1:2] + 1.0
    dinv = lax.rsqrt(deg)
    o = dinv * (a2_ref[0, :N_NODES] + a2_ref[1, :N_NODES] + y2_ref[...]) + b2_ref[...]
    col = lax.broadcasted_iota(jnp.int32, (N_NODES, D_OUT_PAD), 1)
    o = jnp.where(col < D_OUT, o, -1e30)
    m = jnp.max(o, axis=1, keepdims=True)
    e = jnp.exp(o - m)
    lse = jnp.log(jnp.sum(e, axis=1, keepdims=True))
    out_ref[...] = o - m - lse


def kernel(x, edge_index, W1, b1, W2, b2):
    src = edge_index[0]
    dst = edge_index[1]
    pad = E_PAD - N_EDGES
    srcp = jnp.concatenate(
        [src, jnp.zeros((pad,), jnp.int32)]).reshape(N_CHUNKS, B_EDGE)
    dstp = jnp.concatenate(
        [dst, jnp.full((pad,), N_NODES, jnp.int32)]).reshape(N_CHUNKS, B_EDGE)

    z1d = jnp.zeros((DEG_ROWS // NS,), jnp.float32)
    z128 = jnp.zeros((ACC_ROWS // NS, D_HID), jnp.float32)
    z48 = jnp.zeros((ACC_ROWS // NS, D_OUT_PAD), jnp.float32)
    W2p = jnp.pad(W2, ((0, 0), (0, D_OUT_PAD - D_OUT)))
    b2p = jnp.pad(b2, (0, D_OUT_PAD - D_OUT))

    deg_parts = _deg_kernel(dstp, z1d)
    degt = jnp.transpose(deg_parts[:, :N_NODES])

    y1 = pl.pallas_call(
        _tc_pre,
        out_shape=jax.ShapeDtypeStruct((N_NODES, D_HID), jnp.float32),
    )(x, W1, degt)

    a1 = _agg128(y1, srcp, dstp, z128)

    y2 = pl.pallas_call(
        _tc_mid,
        out_shape=jax.ShapeDtypeStruct((N_NODES, D_OUT_PAD), jnp.float32),
    )(a1, y1, degt, W2p, b1)

    a2 = _agg48(y2, srcp, dstp, z48)

    out = pl.pallas_call(
        _tc_post,
        out_shape=jax.ShapeDtypeStruct((N_NODES, D_OUT_PAD), jnp.float32),
    )(a2, y2, degt, b2p)

    return out[:, :D_OUT]


# speedup vs baseline: 11.9561x; 1.1139x over previous
import functools

import jax
import jax.numpy as jnp
from jax import lax
from jax.experimental import pallas as pl
from jax.experimental.pallas import tpu as pltpu
from jax.experimental.pallas import tpu_sc as plsc

N_NODES = 10000
N_EDGES = 320000
D_IN = 128
D_HID = 128
D_OUT = 47
D_OUT_PAD = 48

NC = 2
NS = 16
NW = NC * NS

B_EDGE = 128
E_PAD = 327680
N_CHUNKS = E_PAD // B_EDGE
CPT = N_CHUNKS // NW
SLAB = 40
ACC_ROWS = 10240
DEG_ROWS = 10240

_mesh = plsc.VectorSubcoreMesh(core_axis_name="c", subcore_axis_name="s",
                               num_cores=NC, num_subcores=NS)


def _deg_body(dst_hbm, zeros_hbm, out_hbm, dst_v, ones_v, acc):
    cid = lax.axis_index("c")
    sid = lax.axis_index("s")
    wid = sid * NC + cid
    stripe = DEG_ROWS // NS
    pltpu.sync_copy(dst_hbm.at[pl.ds(wid * CPT, CPT)], dst_v)
    for i in range(B_EDGE // 16):
        ones_v[pl.ds(16 * i, 16)] = jnp.ones((16,), jnp.float32)
    pltpu.sync_copy(zeros_hbm, acc.at[pl.ds(sid * stripe, stripe)])
    plsc.subcore_barrier()

    def body(g, carry):
        pltpu.sync_copy(ones_v, acc.at[dst_v.at[g]], add=True)
        return carry

    lax.fori_loop(0, CPT, body, 0)
    plsc.subcore_barrier()
    pltpu.sync_copy(acc.at[pl.ds(sid * stripe, stripe)],
                    out_hbm.at[cid, pl.ds(sid * stripe, stripe)])


_deg_kernel = functools.partial(
    pl.kernel,
    _deg_body,
    out_type=jax.ShapeDtypeStruct((NC, DEG_ROWS), jnp.float32),
    mesh=_mesh,
    scratch_types=[
        pltpu.VMEM((CPT, B_EDGE), jnp.int32),
        pltpu.VMEM((B_EDGE,), jnp.float32),
        pltpu.VMEM_SHARED((DEG_ROWS,), jnp.float32),
    ],
)()


def _make_agg(d_feat):
    zstripe = ACC_ROWS // NS
    ostripe = ACC_ROWS // NS

    def body(y_hbm, src_hbm, dst_hbm, zeros_hbm, out_hbm,
             src_v, dst_v, buf0, buf1, acc, sem0, sem1):
        cid = lax.axis_index("c")
        sid = lax.axis_index("s")
        wid = sid * NC + cid
        pltpu.sync_copy(zeros_hbm, acc.at[pl.ds(sid * zstripe, zstripe)])
        plsc.subcore_barrier()

        def slab(s, carry):
            base = wid * CPT + s * SLAB
            pltpu.sync_copy(src_hbm.at[pl.ds(base, SLAB)], src_v)
            pltpu.sync_copy(dst_hbm.at[pl.ds(base, SLAB)], dst_v)
            pltpu.async_copy(y_hbm.at[src_v.at[0]], buf0, sem0)

            def pair(k, carry):
                g = 2 * k
                pltpu.async_copy(y_hbm.at[src_v.at[g + 1]], buf1, sem1)
                pltpu.make_async_copy(y_hbm.at[src_v.at[g]], buf0, sem0).wait()
                pltpu.sync_copy(buf0, acc.at[dst_v.at[g]], add=True)

                @pl.when(k < SLAB // 2 - 1)
                def _():
                    pltpu.async_copy(y_hbm.at[src_v.at[g + 2]], buf0, sem0)

                pltpu.make_async_copy(y_hbm.at[src_v.at[g + 1]], buf1,
                                      sem1).wait()
                pltpu.sync_copy(buf1, acc.at[dst_v.at[g + 1]], add=True)
                return carry

            lax.fori_loop(0, SLAB // 2, pair, 0)
            return carry

        lax.fori_loop(0, CPT // SLAB, slab, 0)
        plsc.subcore_barrier()
        pltpu.sync_copy(acc.at[pl.ds(sid * ostripe, ostripe)],
                        out_hbm.at[cid, pl.ds(sid * ostripe, ostripe)])

    return functools.partial(
        pl.kernel,
        body,
        out_type=jax.ShapeDtypeStruct((NC, ACC_ROWS, d_feat), jnp.float32),
        mesh=_mesh,
        scratch_types=[
            pltpu.VMEM((SLAB, B_EDGE), jnp.int32),
            pltpu.VMEM((SLAB, B_EDGE), jnp.int32),
            pltpu.VMEM((B_EDGE, d_feat), jnp.float32),
            pltpu.VMEM((B_EDGE, d_feat), jnp.float32),
            pltpu.VMEM_SHARED((ACC_ROWS, d_feat), jnp.float32),
            pltpu.SemaphoreType.DMA,
            pltpu.SemaphoreType.DMA,
        ],
        compiler_params=pltpu.CompilerParams(
            use_tc_tiling_on_sc=(d_feat % 128 == 0)),
    )()


_agg128 = _make_agg(D_HID)
_agg48 = _make_agg(D_OUT_PAD)


def _tc_pre(x_ref, w1_ref, degt_ref, y_ref):
    deg = degt_ref[:, 0:1] + degt_ref[:, 1:2] + 1.0
    dinv = lax.rsqrt(deg)
    y_ref[...] = jnp.dot(x_ref[...], w1_ref[...],
                         preferred_element_type=jnp.float32) * dinv


def _tc_mid(a1_ref, y1_ref, degt_ref, w2_ref, b1_ref, y2_ref):
    deg = degt_ref[:, 0:1] + degt_ref[:, 1:2] + 1.0
    dinv = lax.rsqrt(deg)
    h = dinv * (a1_ref[0, :N_NODES] + a1_ref[1, :N_NODES] + y1_ref[...]) + b1_ref[...]
    h = jnp.maximum(h, 0.0)
    y2_ref[...] = jnp.dot(h, w2_ref[...],
                          preferred_element_type=jnp.float32) * dinv


def _tc_post(a2_ref, y2_ref, degt_ref, b2_ref, out_ref):
    deg = degt_ref[:, 0:1] + degt_ref[:, 1:2] + 1.0
    dinv = lax.rsqrt(deg)
    o = dinv * (a2_ref[0, :N_NODES] + a2_ref[1, :N_NODES] + y2_ref[...]) + b2_ref[...]
    col = lax.broadcasted_iota(jnp.int32, (N_NODES, D_OUT_PAD), 1)
    o = jnp.where(col < D_OUT, o, -1e30)
    m = jnp.max(o, axis=1, keepdims=True)
    e = jnp.exp(o - m)
    lse = jnp.log(jnp.sum(e, axis=1, keepdims=True))
    out_ref[...] = o - m - lse


def kernel(x, edge_index, W1, b1, W2, b2):
    src = edge_index[0]
    dst = edge_index[1]
    pad = E_PAD - N_EDGES
    srcp = jnp.concatenate(
        [src, jnp.zeros((pad,), jnp.int32)]).reshape(N_CHUNKS, B_EDGE)
    dstp = jnp.concatenate(
        [dst, jnp.full((pad,), N_NODES, jnp.int32)]).reshape(N_CHUNKS, B_EDGE)

    z1d = jnp.zeros((DEG_ROWS // NS,), jnp.float32)
    z128 = jnp.zeros((ACC_ROWS // NS, D_HID), jnp.float32)
    z48 = jnp.zeros((ACC_ROWS // NS, D_OUT_PAD), jnp.float32)
    W2p = jnp.pad(W2, ((0, 0), (0, D_OUT_PAD - D_OUT)))
    b2p = jnp.pad(b2, (0, D_OUT_PAD - D_OUT))

    deg_parts = _deg_kernel(dstp, z1d)
    degt = jnp.transpose(deg_parts[:, :N_NODES])

    y1 = pl.pallas_call(
        _tc_pre,
        out_shape=jax.ShapeDtypeStruct((N_NODES, D_HID), jnp.float32),
    )(x, W1, degt)

    a1 = _agg128(y1, srcp, dstp, z128)

    y2 = pl.pallas_call(
        _tc_mid,
        out_shape=jax.ShapeDtypeStruct((N_NODES, D_OUT_PAD), jnp.float32),
    )(a1, y1, degt, W2p, b1)

    a2 = _agg48(y2, srcp, dstp, z48)

    out = pl.pallas_call(
        _tc_post,
        out_shape=jax.ShapeDtypeStruct((N_NODES, D_OUT_PAD), jnp.float32),
    )(a2, y2, degt, b2p)

    return out[:, :D_OUT]


# speedup vs baseline: 30.1242x; 2.5196x over previous
import functools

import jax
import jax.numpy as jnp
from jax import lax
from jax.experimental import pallas as pl
from jax.experimental.pallas import tpu as pltpu
from jax.experimental.pallas import tpu_sc as plsc

N_NODES = 10000
N_EDGES = 320000
D_IN = 128
D_HID = 128
D_OUT = 47
D_OUT_PAD = 48

NC = 2
NS = 16
NW = NC * NS

B_EDGE = 128
E_PAD = 327680
N_CHUNKS = E_PAD // B_EDGE
CPT = N_CHUNKS // NW
ACC_ROWS = 10240
DEG_ROWS = 10240

_mesh = plsc.VectorSubcoreMesh(core_axis_name="c", subcore_axis_name="s",
                               num_cores=NC, num_subcores=NS)


def _deg_body(dst_hbm, zeros_hbm, out_hbm, dst_v, ones_v, acc):
    cid = lax.axis_index("c")
    sid = lax.axis_index("s")
    wid = sid * NC + cid
    stripe = DEG_ROWS // NS
    pltpu.sync_copy(dst_hbm.at[pl.ds(wid * CPT, CPT)], dst_v)
    for i in range(B_EDGE // 16):
        ones_v[pl.ds(16 * i, 16)] = jnp.ones((16,), jnp.float32)
    pltpu.sync_copy(zeros_hbm, acc.at[pl.ds(sid * stripe, stripe)])
    plsc.subcore_barrier()

    def body(g, carry):
        pltpu.sync_copy(ones_v, acc.at[dst_v.at[g]], add=True)
        return carry

    lax.fori_loop(0, CPT, body, 0)
    plsc.subcore_barrier()
    pltpu.sync_copy(acc.at[pl.ds(sid * stripe, stripe)],
                    out_hbm.at[cid, pl.ds(sid * stripe, stripe)])


_deg_kernel = functools.partial(
    pl.kernel,
    _deg_body,
    out_type=jax.ShapeDtypeStruct((NC, DEG_ROWS), jnp.float32),
    mesh=_mesh,
    scratch_types=[
        pltpu.VMEM((CPT, B_EDGE), jnp.int32),
        pltpu.VMEM((B_EDGE,), jnp.float32),
        pltpu.VMEM_SHARED((DEG_ROWS,), jnp.float32),
    ],
)()


def _make_agg(d_feat, d_sub):
    stripe = ACC_ROWS // NS
    n_pass = d_feat // d_sub

    def body(y_hbm, src_hbm, dst_hbm, zeros_hbm, out_hbm,
             src_v, dst_v, buf0, buf1, table, acc, sem0, sem1):
        cid = lax.axis_index("c")
        sid = lax.axis_index("s")
        wid = sid * NC + cid
        pltpu.sync_copy(src_hbm.at[pl.ds(wid * CPT, CPT)], src_v)
        pltpu.sync_copy(dst_hbm.at[pl.ds(wid * CPT, CPT)], dst_v)
        row0 = sid * stripe

        for h in range(n_pass):
            cols = pl.ds(h * d_sub, d_sub)
            pltpu.sync_copy(y_hbm.at[pl.ds(row0, stripe), cols],
                            table.at[pl.ds(row0, stripe)])
            pltpu.sync_copy(zeros_hbm, acc.at[pl.ds(row0, stripe)])
            plsc.subcore_barrier()

            pltpu.async_copy(table.at[src_v.at[0]], buf0, sem0)

            def pair(k, carry):
                g = 2 * k
                pltpu.async_copy(table.at[src_v.at[g + 1]], buf1, sem1)
                pltpu.make_async_copy(table.at[src_v.at[g]], buf0,
                                      sem0).wait()
                pltpu.sync_copy(buf0, acc.at[dst_v.at[g]], add=True)

                @pl.when(k < CPT // 2 - 1)
                def _():
                    pltpu.async_copy(table.at[src_v.at[g + 2]], buf0, sem0)

                pltpu.make_async_copy(table.at[src_v.at[g + 1]], buf1,
                                      sem1).wait()
                pltpu.sync_copy(buf1, acc.at[dst_v.at[g + 1]], add=True)
                return carry

            lax.fori_loop(0, CPT // 2, pair, 0)
            plsc.subcore_barrier()
            pltpu.sync_copy(acc.at[pl.ds(row0, stripe)],
                            out_hbm.at[cid, pl.ds(row0, stripe), cols])

    return functools.partial(
        pl.kernel,
        body,
        out_type=jax.ShapeDtypeStruct((NC, ACC_ROWS, d_feat), jnp.float32),
        mesh=_mesh,
        scratch_types=[
            pltpu.VMEM((CPT, B_EDGE), jnp.int32),
            pltpu.VMEM((CPT, B_EDGE), jnp.int32),
            pltpu.VMEM((B_EDGE, d_sub), jnp.float32),
            pltpu.VMEM((B_EDGE, d_sub), jnp.float32),
            pltpu.VMEM_SHARED((ACC_ROWS, d_sub), jnp.float32),
            pltpu.VMEM_SHARED((ACC_ROWS, d_sub), jnp.float32),
            pltpu.SemaphoreType.DMA,
            pltpu.SemaphoreType.DMA,
        ],
        compiler_params=pltpu.CompilerParams(use_tc_tiling_on_sc=False),
    )()


_agg128 = _make_agg(D_HID, 64)
_agg48 = _make_agg(D_OUT_PAD, D_OUT_PAD)


def _tc_pre(x_ref, w1_ref, degt_ref, y_ref):
    deg = degt_ref[:, 0:1] + degt_ref[:, 1:2] + 1.0
    dinv = lax.rsqrt(deg)
    y_ref[:N_NODES] = jnp.dot(x_ref[...], w1_ref[...],
                              preferred_element_type=jnp.float32) * dinv
    y_ref[N_NODES:] = jnp.zeros((ACC_ROWS - N_NODES, D_HID), jnp.float32)


def _tc_mid(a1_ref, y1_ref, degt_ref, w2_ref, b1_ref, y2_ref):
    deg = degt_ref[:, 0:1] + degt_ref[:, 1:2] + 1.0
    dinv = lax.rsqrt(deg)
    h = dinv * (a1_ref[0, :N_NODES] + a1_ref[1, :N_NODES]
                + y1_ref[:N_NODES]) + b1_ref[...]
    h = jnp.maximum(h, 0.0)
    y2_ref[:N_NODES] = jnp.dot(h, w2_ref[...],
                               preferred_element_type=jnp.float32) * dinv
    y2_ref[N_NODES:] = jnp.zeros((ACC_ROWS - N_NODES, D_OUT_PAD), jnp.float32)


def _tc_post(a2_ref, y2_ref, degt_ref, b2_ref, out_ref):
    deg = degt_ref[:, 0:1] + degt_ref[:, 1:2] + 1.0
    dinv = lax.rsqrt(deg)
    o = dinv * (a2_ref[0, :N_NODES] + a2_ref[1, :N_NODES]
                + y2_ref[:N_NODES]) + b2_ref[...]
    col = lax.broadcasted_iota(jnp.int32, (N_NODES, D_OUT_PAD), 1)
    o = jnp.where(col < D_OUT, o, -1e30)
    m = jnp.max(o, axis=1, keepdims=True)
    e = jnp.exp(o - m)
    lse = jnp.log(jnp.sum(e, axis=1, keepdims=True))
    out_ref[...] = o - m - lse


def kernel(x, edge_index, W1, b1, W2, b2):
    src = edge_index[0]
    dst = edge_index[1]
    pad = E_PAD - N_EDGES
    srcp = jnp.concatenate(
        [src, jnp.zeros((pad,), jnp.int32)]).reshape(N_CHUNKS, B_EDGE)
    dstp = jnp.concatenate(
        [dst, jnp.full((pad,), N_NODES, jnp.int32)]).reshape(N_CHUNKS, B_EDGE)

    z1d = jnp.zeros((DEG_ROWS // NS,), jnp.float32)
    z128 = jnp.zeros((ACC_ROWS // NS, 64), jnp.float32)
    z48 = jnp.zeros((ACC_ROWS // NS, D_OUT_PAD), jnp.float32)
    W2p = jnp.pad(W2, ((0, 0), (0, D_OUT_PAD - D_OUT)))
    b2p = jnp.pad(b2, (0, D_OUT_PAD - D_OUT))

    deg_parts = _deg_kernel(dstp, z1d)
    degt = jnp.transpose(deg_parts[:, :N_NODES])

    y1 = pl.pallas_call(
        _tc_pre,
        out_shape=jax.ShapeDtypeStruct((ACC_ROWS, D_HID), jnp.float32),
    )(x, W1, degt)

    a1 = _agg128(y1, srcp, dstp, z128)

    y2 = pl.pallas_call(
        _tc_mid,
        out_shape=jax.ShapeDtypeStruct((ACC_ROWS, D_OUT_PAD), jnp.float32),
    )(a1, y1, degt, W2p, b1)

    a2 = _agg48(y2, srcp, dstp, z48)

    out = pl.pallas_call(
        _tc_post,
        out_shape=jax.ShapeDtypeStruct((N_NODES, D_OUT_PAD), jnp.float32),
    )(a2, y2, degt, b2p)

    return out[:, :D_OUT]


# speedup vs baseline: 34.0280x; 1.1296x over previous
import functools

import jax
import jax.numpy as jnp
from jax import lax
from jax.experimental import pallas as pl
from jax.experimental.pallas import tpu as pltpu
from jax.experimental.pallas import tpu_sc as plsc

N_NODES = 10000
N_EDGES = 320000
D_IN = 128
D_HID = 128
D_OUT = 47
D_OUT_PAD = 48

NC = 2
NS = 16
NW = NC * NS

B_EDGE = 125
N_CHUNKS = N_EDGES // B_EDGE
CPT = N_CHUNKS // NW
ACC_ROWS = 10240
DEG_ROWS = 10240

_mesh = plsc.VectorSubcoreMesh(core_axis_name="c", subcore_axis_name="s",
                               num_cores=NC, num_subcores=NS)


def _deg_body(dst_hbm, zeros_hbm, ones_hbm, out_hbm, dst_v, ones_v, acc):
    cid = lax.axis_index("c")
    sid = lax.axis_index("s")
    wid = sid * NC + cid
    stripe = DEG_ROWS // NS
    pltpu.sync_copy(dst_hbm.at[pl.ds(wid * CPT, CPT)], dst_v)
    pltpu.sync_copy(ones_hbm, ones_v)
    pltpu.sync_copy(zeros_hbm, acc.at[pl.ds(sid * stripe, stripe)])
    plsc.subcore_barrier()

    def body(g, carry):
        pltpu.sync_copy(ones_v, acc.at[dst_v.at[g]], add=True)
        return carry

    lax.fori_loop(0, CPT, body, 0)
    plsc.subcore_barrier()
    pltpu.sync_copy(acc.at[pl.ds(sid * stripe, stripe)],
                    out_hbm.at[cid, pl.ds(sid * stripe, stripe)])


_deg_kernel = functools.partial(
    pl.kernel,
    _deg_body,
    out_type=jax.ShapeDtypeStruct((NC, DEG_ROWS), jnp.float32),
    mesh=_mesh,
    scratch_types=[
        pltpu.VMEM((CPT, B_EDGE), jnp.int32),
        pltpu.VMEM((B_EDGE,), jnp.float32),
        pltpu.VMEM_SHARED((DEG_ROWS,), jnp.float32),
    ],
)()


def _make_agg(d_feat, d_sub, slab):
    stripe = ACC_ROWS // NS
    n_pass = d_feat // d_sub
    n_quad = slab // 4

    def body(y_hbm, src_hbm, dst_hbm, zeros_hbm, out_hbm,
             src_v, dst_v, b0, b1, b2, b3, table, acc,
             gs0, gs1, gs2, gs3, ss0, ss1, ss2, ss3):
        cid = lax.axis_index("c")
        sid = lax.axis_index("s")
        wid = sid * NC + cid
        row0 = sid * stripe

        def gather(g, buf, sem):
            pltpu.async_copy(table.at[src_v.at[g]], buf, sem)

        def gather_wait(g, buf, sem):
            pltpu.make_async_copy(table.at[src_v.at[g]], buf, sem).wait()

        def scat(g, buf, sem):
            pltpu.async_copy(buf, acc.at[dst_v.at[g]], sem, add=True)

        def scat_wait(g, buf, sem):
            pltpu.make_async_copy(buf, acc.at[dst_v.at[g]], sem).wait()

        for h in range(n_pass):
            cols = pl.ds(h * d_sub, d_sub)
            pltpu.sync_copy(y_hbm.at[pl.ds(row0, stripe), cols],
                            table.at[pl.ds(row0, stripe)])
            pltpu.sync_copy(zeros_hbm, acc.at[pl.ds(row0, stripe)])
            plsc.subcore_barrier()

            def do_slab(s, carry):
                base = wid * CPT + s * slab
                pltpu.sync_copy(src_hbm.at[pl.ds(base, slab)], src_v)
                pltpu.sync_copy(dst_hbm.at[pl.ds(base, slab)], dst_v)
                gather(0, b0, gs0)
                gather(1, b1, gs1)

                def quad(k, carry):
                    g = 4 * k
                    gather_wait(g, b0, gs0)
                    scat(g, b0, ss0)

                    @pl.when(k > 0)
                    def _():
                        scat_wait(g - 2, b2, ss2)
                    gather(g + 2, b2, gs2)

                    gather_wait(g + 1, b1, gs1)
                    scat(g + 1, b1, ss1)

                    @pl.when(k > 0)
                    def _():
                        scat_wait(g - 1, b3, ss3)
                    gather(g + 3, b3, gs3)

                    gather_wait(g + 2, b2, gs2)
                    scat(g + 2, b2, ss2)

                    @pl.when(k < n_quad - 1)
                    def _():
                        scat_wait(g, b0, ss0)
                        gather(g + 4, b0, gs0)

                    gather_wait(g + 3, b3, gs3)
                    scat(g + 3, b3, ss3)

                    @pl.when(k < n_quad - 1)
                    def _():
                        scat_wait(g + 1, b1, ss1)
                        gather(g + 5, b1, gs1)

                    return carry

                lax.fori_loop(0, n_quad, quad, 0)
                scat_wait(slab - 4, b0, ss0)
                scat_wait(slab - 3, b1, ss1)
                scat_wait(slab - 2, b2, ss2)
                scat_wait(slab - 1, b3, ss3)
                return carry

            lax.fori_loop(0, CPT // slab, do_slab, 0)
            plsc.subcore_barrier()
            pltpu.sync_copy(acc.at[pl.ds(row0, stripe)],
                            out_hbm.at[cid, pl.ds(row0, stripe), cols])

    return functools.partial(
        pl.kernel,
        body,
        out_type=jax.ShapeDtypeStruct((NC, ACC_ROWS, d_feat), jnp.float32),
        mesh=_mesh,
        scratch_types=(
            [pltpu.VMEM((slab, B_EDGE), jnp.int32),
             pltpu.VMEM((slab, B_EDGE), jnp.int32)]
            + [pltpu.VMEM((B_EDGE, d_sub), jnp.float32)] * 4
            + [pltpu.VMEM_SHARED((ACC_ROWS, d_sub), jnp.float32)] * 2
            + [pltpu.SemaphoreType.DMA] * 8
        ),
        compiler_params=pltpu.CompilerParams(use_tc_tiling_on_sc=False),
    )()


_agg128 = _make_agg(D_HID, 64, 40)
_agg48 = _make_agg(D_OUT_PAD, D_OUT_PAD, 80)


def _tc_pre(x_ref, w1_ref, degt_ref, y_ref):
    deg = degt_ref[:, 0:1] + degt_ref[:, 1:2] + 1.0
    dinv = lax.rsqrt(deg)
    y_ref[:N_NODES] = jnp.dot(x_ref[...], w1_ref[...],
                              preferred_element_type=jnp.float32) * dinv
    y_ref[N_NODES:] = jnp.zeros((ACC_ROWS - N_NODES, D_HID), jnp.float32)


def _tc_mid(a1_ref, y1_ref, degt_ref, w2_ref, b1_ref, y2_ref):
    deg = degt_ref[:, 0:1] + degt_ref[:, 1:2] + 1.0
    dinv = lax.rsqrt(deg)
    h = dinv * (a1_ref[0, :N_NODES] + a1_ref[1, :N_NODES]
                + y1_ref[:N_NODES]) + b1_ref[...]
    h = jnp.maximum(h, 0.0)
    y2_ref[:N_NODES] = jnp.dot(h, w2_ref[...],
                               preferred_element_type=jnp.float32) * dinv
    y2_ref[N_NODES:] = jnp.zeros((ACC_ROWS - N_NODES, D_OUT_PAD), jnp.float32)


def _tc_post(a2_ref, y2_ref, degt_ref, b2_ref, out_ref):
    deg = degt_ref[:, 0:1] + degt_ref[:, 1:2] + 1.0
    dinv = lax.rsqrt(deg)
    o = dinv * (a2_ref[0, :N_NODES] + a2_ref[1, :N_NODES]
                + y2_ref[:N_NODES]) + b2_ref[...]
    col = lax.broadcasted_iota(jnp.int32, (N_NODES, D_OUT_PAD), 1)
    o = jnp.where(col < D_OUT, o, -1e30)
    m = jnp.max(o, axis=1, keepdims=True)
    e = jnp.exp(o - m)
    lse = jnp.log(jnp.sum(e, axis=1, keepdims=True))
    out_ref[...] = o - m - lse


def kernel(x, edge_index, W1, b1, W2, b2):
    srcp = edge_index[0].reshape(N_CHUNKS, B_EDGE)
    dstp = edge_index[1].reshape(N_CHUNKS, B_EDGE)
    ones = jnp.ones((B_EDGE,), jnp.float32)

    z1d = jnp.zeros((DEG_ROWS // NS,), jnp.float32)
    z128 = jnp.zeros((ACC_ROWS // NS, 64), jnp.float32)
    z48 = jnp.zeros((ACC_ROWS // NS, D_OUT_PAD), jnp.float32)
    W2p = jnp.pad(W2, ((0, 0), (0, D_OUT_PAD - D_OUT)))
    b2p = jnp.pad(b2, (0, D_OUT_PAD - D_OUT))

    deg_parts = _deg_kernel(dstp, z1d, ones)
    degt = jnp.transpose(deg_parts[:, :N_NODES])

    y1 = pl.pallas_call(
        _tc_pre,
        out_shape=jax.ShapeDtypeStruct((ACC_ROWS, D_HID), jnp.float32),
    )(x, W1, degt)

    a1 = _agg128(y1, srcp, dstp, z128)

    y2 = pl.pallas_call(
        _tc_mid,
        out_shape=jax.ShapeDtypeStruct((ACC_ROWS, D_OUT_PAD), jnp.float32),
    )(a1, y1, degt, W2p, b1)

    a2 = _agg48(y2, srcp, dstp, z48)

    out = pl.pallas_call(
        _tc_post,
        out_shape=jax.ShapeDtypeStruct((N_NODES, D_OUT_PAD), jnp.float32),
    )(a2, y2, degt, b2p)

    return out[:, :D_OUT]


# speedup vs baseline: 35.1171x; 1.0320x over previous
import functools

import jax
import jax.numpy as jnp
from jax import lax
from jax.experimental import pallas as pl
from jax.experimental.pallas import tpu as pltpu
from jax.experimental.pallas import tpu_sc as plsc

N_NODES = 10000
N_EDGES = 320000
D_IN = 128
D_HID = 128
D_OUT = 47
D_OUT_PAD = 48

NC = 2
NS = 16
NW = NC * NS

B_EDGE = 125
N_CHUNKS = N_EDGES // B_EDGE
CPT = N_CHUNKS // NW
ACC_ROWS = 10240
DEG_ROWS = 10240

_mesh = plsc.VectorSubcoreMesh(core_axis_name="c", subcore_axis_name="s",
                               num_cores=NC, num_subcores=NS)


def _deg_body(edge_hbm, zeros_hbm, ones_hbm, out_hbm, dst_v, ones_v, acc):
    cid = lax.axis_index("c")
    sid = lax.axis_index("s")
    wid = sid * NC + cid
    stripe = DEG_ROWS // NS
    pltpu.sync_copy(edge_hbm.at[1, pl.ds(wid * CPT, CPT)], dst_v)
    pltpu.sync_copy(ones_hbm, ones_v)
    pltpu.sync_copy(zeros_hbm, acc.at[pl.ds(sid * stripe, stripe)])
    plsc.subcore_barrier()

    def body(g, carry):
        pltpu.sync_copy(ones_v, acc.at[dst_v.at[g]], add=True)
        return carry

    lax.fori_loop(0, CPT, body, 0)
    plsc.subcore_barrier()
    pltpu.sync_copy(acc.at[pl.ds(sid * stripe, stripe)],
                    out_hbm.at[cid, pl.ds(sid * stripe, stripe)])


_deg_kernel = functools.partial(
    pl.kernel,
    _deg_body,
    out_type=jax.ShapeDtypeStruct((NC, DEG_ROWS), jnp.float32),
    mesh=_mesh,
    scratch_types=[
        pltpu.VMEM((CPT, B_EDGE), jnp.int32),
        pltpu.VMEM((B_EDGE,), jnp.float32),
        pltpu.VMEM_SHARED((DEG_ROWS,), jnp.float32),
    ],
)()


def _make_agg(d_feat, d_sub, slab):
    stripe = ACC_ROWS // NS
    n_pass = d_feat // d_sub
    n_quad = slab // 4

    def body(y_hbm, edge_hbm, zeros_hbm, out_hbm,
             src_v, dst_v, b0, b1, b2, b3, table, acc,
             gs0, gs1, gs2, gs3, ss0, ss1, ss2, ss3):
        cid = lax.axis_index("c")
        sid = lax.axis_index("s")
        wid = sid * NC + cid
        row0 = sid * stripe

        def gather(g, buf, sem):
            pltpu.async_copy(table.at[src_v.at[g]], buf, sem)

        def gather_wait(g, buf, sem):
            pltpu.make_async_copy(table.at[src_v.at[g]], buf, sem).wait()

        def scat(g, buf, sem):
            pltpu.async_copy(buf, acc.at[dst_v.at[g]], sem, add=True)

        def scat_wait(g, buf, sem):
            pltpu.make_async_copy(buf, acc.at[dst_v.at[g]], sem).wait()

        for h in range(n_pass):
            cols = pl.ds(h * d_sub, d_sub)
            pltpu.sync_copy(y_hbm.at[pl.ds(row0, stripe), cols],
                            table.at[pl.ds(row0, stripe)])
            pltpu.sync_copy(zeros_hbm, acc.at[pl.ds(row0, stripe)])
            plsc.subcore_barrier()

            def do_slab(s, carry):
                base = wid * CPT + s * slab
                pltpu.sync_copy(edge_hbm.at[0, pl.ds(base, slab)], src_v)
                pltpu.sync_copy(edge_hbm.at[1, pl.ds(base, slab)], dst_v)
                gather(0, b0, gs0)
                gather(1, b1, gs1)

                def quad(k, carry):
                    g = 4 * k
                    gather_wait(g, b0, gs0)
                    scat(g, b0, ss0)

                    @pl.when(k > 0)
                    def _():
                        scat_wait(g - 2, b2, ss2)
                    gather(g + 2, b2, gs2)

                    gather_wait(g + 1, b1, gs1)
                    scat(g + 1, b1, ss1)

                    @pl.when(k > 0)
                    def _():
                        scat_wait(g - 1, b3, ss3)
                    gather(g + 3, b3, gs3)

                    gather_wait(g + 2, b2, gs2)
                    scat(g + 2, b2, ss2)

                    @pl.when(k < n_quad - 1)
                    def _():
                        scat_wait(g, b0, ss0)
                        gather(g + 4, b0, gs0)

                    gather_wait(g + 3, b3, gs3)
                    scat(g + 3, b3, ss3)

                    @pl.when(k < n_quad - 1)
                    def _():
                        scat_wait(g + 1, b1, ss1)
                        gather(g + 5, b1, gs1)

                    return carry

                lax.fori_loop(0, n_quad, quad, 0)
                scat_wait(slab - 4, b0, ss0)
                scat_wait(slab - 3, b1, ss1)
                scat_wait(slab - 2, b2, ss2)
                scat_wait(slab - 1, b3, ss3)
                return carry

            lax.fori_loop(0, CPT // slab, do_slab, 0)
            plsc.subcore_barrier()
            pltpu.sync_copy(acc.at[pl.ds(row0, stripe)],
                            out_hbm.at[cid, pl.ds(row0, stripe), cols])

    return functools.partial(
        pl.kernel,
        body,
        out_type=jax.ShapeDtypeStruct((NC, ACC_ROWS, d_feat), jnp.float32),
        mesh=_mesh,
        scratch_types=(
            [pltpu.VMEM((slab, B_EDGE), jnp.int32),
             pltpu.VMEM((slab, B_EDGE), jnp.int32)]
            + [pltpu.VMEM((B_EDGE, d_sub), jnp.float32)] * 4
            + [pltpu.VMEM_SHARED((ACC_ROWS, d_sub), jnp.float32)] * 2
            + [pltpu.SemaphoreType.DMA] * 8
        ),
        compiler_params=pltpu.CompilerParams(use_tc_tiling_on_sc=False),
    )()


_agg128 = _make_agg(D_HID, 64, 40)
_agg48 = _make_agg(D_OUT_PAD, D_OUT_PAD, 80)


def _tc_pre(x_ref, w1_ref, degt_ref, y_ref):
    deg = degt_ref[:, 0:1] + degt_ref[:, 1:2] + 1.0
    dinv = lax.rsqrt(deg)
    y_ref[:N_NODES] = jnp.dot(x_ref[...], w1_ref[...],
                              preferred_element_type=jnp.float32) * dinv
    y_ref[N_NODES:] = jnp.zeros((ACC_ROWS - N_NODES, D_HID), jnp.float32)


def _tc_mid(a1_ref, y1_ref, degt_ref, w2_ref, b1_ref, y2_ref):
    deg = degt_ref[:, 0:1] + degt_ref[:, 1:2] + 1.0
    dinv = lax.rsqrt(deg)
    h = dinv * (a1_ref[0, :N_NODES] + a1_ref[1, :N_NODES]
                + y1_ref[:N_NODES]) + b1_ref[...]
    h = jnp.maximum(h, 0.0)
    y2_ref[:N_NODES] = jnp.dot(h, w2_ref[...],
                               preferred_element_type=jnp.float32) * dinv
    y2_ref[N_NODES:] = jnp.zeros((ACC_ROWS - N_NODES, D_OUT_PAD), jnp.float32)


def _tc_post(a2_ref, y2_ref, degt_ref, b2_ref, out_ref):
    deg = degt_ref[:, 0:1] + degt_ref[:, 1:2] + 1.0
    dinv = lax.rsqrt(deg)
    o = dinv * (a2_ref[0, :N_NODES] + a2_ref[1, :N_NODES]
                + y2_ref[:N_NODES]) + b2_ref[...]
    col = lax.broadcasted_iota(jnp.int32, (N_NODES, D_OUT_PAD), 1)
    o = jnp.where(col < D_OUT, o, -1e30)
    m = jnp.max(o, axis=1, keepdims=True)
    e = jnp.exp(o - m)
    lse = jnp.log(jnp.sum(e, axis=1, keepdims=True))
    out_ref[...] = (o - m - lse)[:, :D_OUT]


def kernel(x, edge_index, W1, b1, W2, b2):
    edges = edge_index.reshape(2, N_CHUNKS, B_EDGE)
    ones = jnp.ones((B_EDGE,), jnp.float32)

    z1d = jnp.zeros((DEG_ROWS // NS,), jnp.float32)
    z128 = jnp.zeros((ACC_ROWS // NS, 64), jnp.float32)
    z48 = jnp.zeros((ACC_ROWS // NS, D_OUT_PAD), jnp.float32)
    W2p = jnp.pad(W2, ((0, 0), (0, D_OUT_PAD - D_OUT)))
    b2p = jnp.pad(b2, (0, D_OUT_PAD - D_OUT))

    deg_parts = _deg_kernel(edges, z1d, ones)
    degt = jnp.transpose(deg_parts[:, :N_NODES])

    y1 = pl.pallas_call(
        _tc_pre,
        out_shape=jax.ShapeDtypeStruct((ACC_ROWS, D_HID), jnp.float32),
    )(x, W1, degt)

    a1 = _agg128(y1, edges, z128)

    y2 = pl.pallas_call(
        _tc_mid,
        out_shape=jax.ShapeDtypeStruct((ACC_ROWS, D_OUT_PAD), jnp.float32),
    )(a1, y1, degt, W2p, b1)

    a2 = _agg48(y2, edges, z48)

    return pl.pallas_call(
        _tc_post,
        out_shape=jax.ShapeDtypeStruct((N_NODES, D_OUT), jnp.float32),
    )(a2, y2, degt, b2p)


# speedup vs baseline: 36.7339x; 1.0460x over previous
import functools

import jax
import jax.numpy as jnp
from jax import lax
from jax.experimental import pallas as pl
from jax.experimental.pallas import tpu as pltpu
from jax.experimental.pallas import tpu_sc as plsc

N_NODES = 10000
N_EDGES = 320000
D_IN = 128
D_HID = 128
D_OUT = 47
D_OUT_PAD = 48

NC = 2
NS = 16
NW = NC * NS

B_EDGE = 125
N_CHUNKS = N_EDGES // B_EDGE
CPT = N_CHUNKS // NW
ACC_ROWS = 10240
DEG_ROWS = 10240

_mesh = plsc.VectorSubcoreMesh(core_axis_name="c", subcore_axis_name="s",
                               num_cores=NC, num_subcores=NS)


def _deg_body(edge_hbm, zeros_hbm, ones_hbm, out_hbm, dst_v, ones_v, acc):
    cid = lax.axis_index("c")
    sid = lax.axis_index("s")
    wid = sid * NC + cid
    stripe = DEG_ROWS // NS
    pltpu.sync_copy(edge_hbm.at[1, pl.ds(wid * CPT, CPT)], dst_v)
    pltpu.sync_copy(ones_hbm, ones_v)
    pltpu.sync_copy(zeros_hbm, acc.at[pl.ds(sid * stripe, stripe)])
    plsc.subcore_barrier()

    def body(g, carry):
        pltpu.sync_copy(ones_v, acc.at[dst_v.at[g]], add=True)
        return carry

    lax.fori_loop(0, CPT, body, 0)
    plsc.subcore_barrier()
    pltpu.sync_copy(acc.at[pl.ds(sid * stripe, stripe)],
                    out_hbm.at[cid, pl.ds(sid * stripe, stripe)])


_deg_kernel = functools.partial(
    pl.kernel,
    _deg_body,
    out_type=jax.ShapeDtypeStruct((NC, DEG_ROWS), jnp.float32),
    mesh=_mesh,
    scratch_types=[
        pltpu.VMEM((CPT, B_EDGE), jnp.int32),
        pltpu.VMEM((B_EDGE,), jnp.float32),
        pltpu.VMEM_SHARED((DEG_ROWS,), jnp.float32),
    ],
)()


def _make_agg(d_feat, d_sub, slab):
    stripe = ACC_ROWS // NS
    n_pass = d_feat // d_sub
    n_quad = slab // 4

    def body(y_hbm, edge_hbm, zeros_hbm, out_hbm,
             src_v, dst_v, b0, b1, b2, b3, table, acc,
             gs0, gs1, gs2, gs3, ss0, ss1, ss2, ss3):
        cid = lax.axis_index("c")
        sid = lax.axis_index("s")
        wid = sid * NC + cid
        row0 = sid * stripe

        def gather(g, buf, sem):
            pltpu.async_copy(table.at[src_v.at[g]], buf, sem)

        def gather_wait(g, buf, sem):
            pltpu.make_async_copy(table.at[src_v.at[g]], buf, sem).wait()

        def scat(g, buf, sem):
            pltpu.async_copy(buf, acc.at[dst_v.at[g]], sem, add=True)

        def scat_wait(g, buf, sem):
            pltpu.make_async_copy(buf, acc.at[dst_v.at[g]], sem).wait()

        for h in range(n_pass):
            cols = pl.ds(h * d_sub, d_sub)
            pltpu.sync_copy(y_hbm.at[pl.ds(row0, stripe), cols],
                            table.at[pl.ds(row0, stripe)])
            pltpu.sync_copy(zeros_hbm, acc.at[pl.ds(row0, stripe)])
            plsc.subcore_barrier()

            def do_slab(s, carry):
                base = wid * CPT + s * slab
                pltpu.sync_copy(edge_hbm.at[0, pl.ds(base, slab)], src_v)
                pltpu.sync_copy(edge_hbm.at[1, pl.ds(base, slab)], dst_v)
                gather(0, b0, gs0)
                gather(1, b1, gs1)

                def quad(k, carry):
                    g = 4 * k
                    gather_wait(g, b0, gs0)
                    scat(g, b0, ss0)

                    @pl.when(k > 0)
                    def _():
                        scat_wait(g - 2, b2, ss2)
                    gather(g + 2, b2, gs2)

                    gather_wait(g + 1, b1, gs1)
                    scat(g + 1, b1, ss1)

                    @pl.when(k > 0)
                    def _():
                        scat_wait(g - 1, b3, ss3)
                    gather(g + 3, b3, gs3)

                    gather_wait(g + 2, b2, gs2)
                    scat(g + 2, b2, ss2)

                    @pl.when(k < n_quad - 1)
                    def _():
                        scat_wait(g, b0, ss0)
                        gather(g + 4, b0, gs0)

                    gather_wait(g + 3, b3, gs3)
                    scat(g + 3, b3, ss3)

                    @pl.when(k < n_quad - 1)
                    def _():
                        scat_wait(g + 1, b1, ss1)
                        gather(g + 5, b1, gs1)

                    return carry

                lax.fori_loop(0, n_quad, quad, 0)
                scat_wait(slab - 4, b0, ss0)
                scat_wait(slab - 3, b1, ss1)
                scat_wait(slab - 2, b2, ss2)
                scat_wait(slab - 1, b3, ss3)
                return carry

            lax.fori_loop(0, CPT // slab, do_slab, 0)
            plsc.subcore_barrier()
            pltpu.sync_copy(acc.at[pl.ds(row0, stripe)],
                            out_hbm.at[cid, pl.ds(row0, stripe), cols])

    return functools.partial(
        pl.kernel,
        body,
        out_type=jax.ShapeDtypeStruct((NC, ACC_ROWS, D_HID), jnp.float32),
        mesh=_mesh,
        scratch_types=(
            [pltpu.VMEM((slab, B_EDGE), jnp.int32),
             pltpu.VMEM((slab, B_EDGE), jnp.int32)]
            + [pltpu.VMEM((B_EDGE, d_sub), jnp.float32)] * 4
            + [pltpu.VMEM_SHARED((ACC_ROWS, d_sub), jnp.float32)] * 2
            + [pltpu.SemaphoreType.DMA] * 8
        ),
        compiler_params=pltpu.CompilerParams(use_tc_tiling_on_sc=False),
    )()


_agg128 = _make_agg(D_HID, 64, 40)
_agg48 = _make_agg(D_OUT_PAD, D_OUT_PAD, 80)


def _tc_pre(x_ref, w1_ref, degt_ref, y_ref):
    deg = degt_ref[:, 0:1] + degt_ref[:, 1:2] + 1.0
    dinv = lax.rsqrt(deg)
    y_ref[:N_NODES] = jnp.dot(x_ref[...], w1_ref[...],
                              preferred_element_type=jnp.float32) * dinv
    y_ref[N_NODES:] = jnp.zeros((ACC_ROWS - N_NODES, D_HID), jnp.float32)


def _tc_mid(a1_ref, y1_ref, degt_ref, w2_ref, b1_ref, y2_ref):
    deg = degt_ref[:, 0:1] + degt_ref[:, 1:2] + 1.0
    dinv = lax.rsqrt(deg)
    h = dinv * (a1_ref[0, :N_NODES] + a1_ref[1, :N_NODES]
                + y1_ref[:N_NODES]) + b1_ref[...]
    h = jnp.maximum(h, 0.0)
    y2_ref[:N_NODES] = jnp.dot(h, w2_ref[...],
                               preferred_element_type=jnp.float32) * dinv
    y2_ref[N_NODES:] = jnp.zeros((ACC_ROWS - N_NODES, D_HID), jnp.float32)


def _tc_post(a2_ref, y2_ref, degt_ref, b2_ref, out_ref):
    deg = degt_ref[:, 0:1] + degt_ref[:, 1:2] + 1.0
    dinv = lax.rsqrt(deg)
    o = dinv * (a2_ref[0, :N_NODES, :D_OUT_PAD]
                + a2_ref[1, :N_NODES, :D_OUT_PAD]
                + y2_ref[:N_NODES, :D_OUT_PAD]) + b2_ref[...]
    col = lax.broadcasted_iota(jnp.int32, (N_NODES, D_OUT_PAD), 1)
    o = jnp.where(col < D_OUT, o, -1e30)
    m = jnp.max(o, axis=1, keepdims=True)
    e = jnp.exp(o - m)
    lse = jnp.log(jnp.sum(e, axis=1, keepdims=True))
    out_ref[...] = (o - m - lse)[:, :D_OUT]


def kernel(x, edge_index, W1, b1, W2, b2):
    edges = edge_index.reshape(2, N_CHUNKS, B_EDGE)
    ones = jnp.ones((B_EDGE,), jnp.float32)

    z1d = jnp.zeros((DEG_ROWS // NS,), jnp.float32)
    z128 = jnp.zeros((ACC_ROWS // NS, 64), jnp.float32)
    z48 = jnp.zeros((ACC_ROWS // NS, D_OUT_PAD), jnp.float32)
    W2p = jnp.pad(W2, ((0, 0), (0, D_HID - D_OUT)))
    b2p = jnp.pad(b2, (0, D_OUT_PAD - D_OUT))

    deg_parts = _deg_kernel(edges, z1d, ones)
    degt = jnp.transpose(deg_parts[:, :N_NODES])

    y1 = pl.pallas_call(
        _tc_pre,
        out_shape=jax.ShapeDtypeStruct((ACC_ROWS, D_HID), jnp.float32),
    )(x, W1, degt)

    a1 = _agg128(y1, edges, z128)

    y2 = pl.pallas_call(
        _tc_mid,
        out_shape=jax.ShapeDtypeStruct((ACC_ROWS, D_HID), jnp.float32),
    )(a1, y1, degt, W2p, b1)

    a2 = _agg48(y2, edges, z48)

    return pl.pallas_call(
        _tc_post,
        out_shape=jax.ShapeDtypeStruct((N_NODES, D_OUT), jnp.float32),
    )(a2, y2, degt, b2p)
